# baseline (speedup 1.0000x reference)
import functools

import jax
import jax.numpy as jnp
from jax import lax
from jax.experimental import pallas as pl
from jax.experimental.pallas import tpu as pltpu
from jax.experimental.pallas import tpu_sc as plsc

HIDDEN = 768
FFN = 3072
E = 8
N_TOK = 1024
NA = 2 * N_TOK
TBR = 128
NS = 24
NW = 32
GPT = (NS * TBR) // NW
TPT = N_TOK // NW


def _router_body(x_ref, wg_ref, ei_ref, ew_ref):
    x = x_ref[...]
    logits = lax.dot_general(x, wg_ref[...], (((1,), (1,)), ((), ())),
                             preferred_element_type=jnp.float32)
    m = jax.nn.softmax(logits, axis=-1)
    i1 = jnp.argmax(m, axis=-1).astype(jnp.int32)
    w1 = jnp.max(m, axis=-1)
    col = lax.broadcasted_iota(jnp.int32, m.shape, 1)
    m2 = jnp.where(col == i1[:, None], -jnp.inf, m)
    i2 = jnp.argmax(m2, axis=-1).astype(jnp.int32)
    w2 = jnp.max(m2, axis=-1)
    d = w1 + w2
    ei_ref[0, :] = i1
    ei_ref[1, :] = i2
    ew_ref[0, :] = w1 / d
    ew_ref[1, :] = w2 / d


def _router(x, Wg):
    return pl.pallas_call(
        _router_body,
        out_shape=(
            jax.ShapeDtypeStruct((2, N_TOK), jnp.int32),
            jax.ShapeDtypeStruct((2, N_TOK), jnp.float32),
        ),
    )(x, Wg)


def _sort_body(eid_hbm, stok_hbm, pos_hbm, sexp_hbm, nb_hbm,
               eid_v, stok_v, sa_v, pos_v, sexp_v, nb_v):
    cid = lax.axis_index("c")
    sid = lax.axis_index("s")

    @pl.when(jnp.logical_and(cid == 0, sid == 0))
    def _():
        pltpu.sync_copy(eid_hbm, eid_v)
        ioto = lax.iota(jnp.int32, 16)
        zeros = jnp.zeros((16,), jnp.int32)

        def initloop(i, _):
            stok_v[pl.ds(i * 16, 16)] = zeros
            sa_v[pl.ds(i * 16, 16)] = zeros + NA
            return 0
        lax.fori_loop(0, (NS * TBR + 16) // 16, initloop, 0)

        def histloop(i, cnts):
            v = eid_v[pl.ds(i * 16, 16)]
            return tuple(
                cnts[e] + plsc.all_reduce_population_count(v == e)
                for e in range(E))
        cnts = lax.fori_loop(
            0, NA // 16, histloop,
            tuple(jnp.zeros((16,), jnp.int32) for _ in range(E)))
        counts = [cnts[e][0] for e in range(E)]
        nbs = [(counts[e] + (TBR - 1)) >> 7 for e in range(E)]
        starts = []
        acc = jnp.int32(0)
        for e in range(E):
            starts.append(acc)
            acc = acc + nbs[e]
        num_blocks = acc
        ends = [starts[e] + nbs[e] for e in range(E)]

        for r in range(2):
            sl = ioto + r * 16
            ecnt = zeros
            for e in range(E):
                ecnt = ecnt + jnp.where(ends[e] <= sl, 1, 0)
            sexp_v[pl.ds(r * 16, 16)] = jnp.minimum(ecnt, E - 1)
        nb_v[...] = zeros + num_blocks

        for e in range(E):
            def p2loop(i, c, e=e):
                v = eid_v[pl.ds(i * 16, 16)]
                a_vec = ioto + i * 16
                m = v == e
                plsc.store_compressed(stok_v.at[pl.ds(c, 16)],
                                      a_vec & (N_TOK - 1), mask=m)
                plsc.store_compressed(sa_v.at[pl.ds(c, 16)], a_vec, mask=m)
                return c + plsc.all_reduce_population_count(m)[0]
            lax.fori_loop(0, NA // 16, p2loop, starts[e] * TBR)

        def invloop(j, _):
            av = sa_v[pl.ds(j * 16, 16)]
            plsc.store_scatter(pos_v, [av], ioto + j * 16)
            return 0
        lax.fori_loop(0, (NS * TBR) // 16, invloop, 0)

        pltpu.sync_copy(stok_v.at[pl.ds(0, NS * TBR)], stok_hbm)
        pltpu.sync_copy(pos_v.at[pl.ds(0, NA)], pos_hbm)
        pltpu.sync_copy(sexp_v, sexp_hbm)
        pltpu.sync_copy(nb_v, nb_hbm)


def _sort(eid):
    return pl.kernel(
        _sort_body,
        out_type=(
            jax.ShapeDtypeStruct((NS * TBR,), jnp.int32),
            jax.ShapeDtypeStruct((NA,), jnp.int32),
            jax.ShapeDtypeStruct((32,), jnp.int32),
            jax.ShapeDtypeStruct((16,), jnp.int32),
        ),
        mesh=plsc.VectorSubcoreMesh(core_axis_name="c", subcore_axis_name="s"),
        compiler_params=pltpu.CompilerParams(needs_layout_passes=False),
        scratch_types=[
            pltpu.VMEM((NA,), jnp.int32),
            pltpu.VMEM((NS * TBR + 16,), jnp.int32),
            pltpu.VMEM((NS * TBR + 16,), jnp.int32),
            pltpu.VMEM((NA + 16,), jnp.int32),
            pltpu.VMEM((32,), jnp.int32),
            pltpu.VMEM((16,), jnp.int32),
        ],
    )(eid)


def _gather_body(stok_hbm, x_hbm, xs_hbm, idx_v, rows_v, sem):
    wid = lax.axis_index("s") * 2 + lax.axis_index("c")
    base = wid * GPT
    pltpu.sync_copy(stok_hbm.at[pl.ds(base, GPT)], idx_v)
    pltpu.async_copy(x_hbm.at[idx_v], rows_v, sem).wait()
    pltpu.sync_copy(rows_v, xs_hbm.at[pl.ds(base, GPT)])


def _gather(stok, x):
    return pl.kernel(
        _gather_body,
        out_type=jax.ShapeDtypeStruct((NS * TBR, HIDDEN), jnp.float32),
        mesh=plsc.VectorSubcoreMesh(core_axis_name="c", subcore_axis_name="s"),
        scratch_types=[
            pltpu.VMEM((GPT,), jnp.int32),
            pltpu.VMEM((GPT, HIDDEN), jnp.float32),
            pltpu.SemaphoreType.DMA,
        ],
    )(stok, x)


def _mlp_body(sexp_ref, nb_ref, xs_ref, w1_ref, w2_ref, w3_ref, ys_ref):
    s = pl.program_id(0)

    @pl.when(s < nb_ref[0])
    def _():
        x = xs_ref[...]
        h1 = jnp.maximum(
            lax.dot_general(x, w1_ref[0], (((1,), (1,)), ((), ())),
                            preferred_element_type=jnp.float32), 0.0)
        h2 = jnp.maximum(
            lax.dot_general(h1, w2_ref[0], (((1,), (1,)), ((), ())),
                            preferred_element_type=jnp.float32), 0.0)
        ys_ref[...] = lax.dot_general(h2, w3_ref[0], (((1,), (1,)), ((), ())),
                                      preferred_element_type=jnp.float32)


def _mlp(sexp, nbv, xs, W1, W2, W3):
    grid_spec = pltpu.PrefetchScalarGridSpec(
        num_scalar_prefetch=2,
        grid=(NS,),
        in_specs=[
            pl.BlockSpec((TBR, HIDDEN), lambda s, se, nb: (s, 0)),
            pl.BlockSpec((1, HIDDEN, HIDDEN), lambda s, se, nb: (se[s], 0, 0)),
            pl.BlockSpec((1, HIDDEN, HIDDEN), lambda s, se, nb: (se[s], 0, 0)),
            pl.BlockSpec((1, FFN, HIDDEN), lambda s, se, nb: (se[s], 0, 0)),
        ],
        out_specs=pl.BlockSpec((TBR, FFN), lambda s, se, nb: (s, 0)),
    )
    return pl.pallas_call(
        _mlp_body,
        grid_spec=grid_spec,
        out_shape=jax.ShapeDtypeStruct((NS * TBR, FFN), jnp.float32),
        compiler_params=pltpu.CompilerParams(
            dimension_semantics=("arbitrary",),
        ),
    )(sexp, nbv, xs, W1, W2, W3)


def _combine_body(pos_hbm, wgt_hbm, ys_hbm, out_hbm,
                  p0_v, p1_v, w0_v, w1_v, rowsA, rowsB, acc_v, sem):
    wid = lax.axis_index("s") * 2 + lax.axis_index("c")
    tb = wid * TPT
    pltpu.sync_copy(pos_hbm.at[pl.ds(tb, TPT)], p0_v)
    pltpu.sync_copy(pos_hbm.at[pl.ds(N_TOK + tb, TPT)], p1_v)
    pltpu.sync_copy(wgt_hbm.at[pl.ds(tb, TPT)], w0_v)
    pltpu.sync_copy(wgt_hbm.at[pl.ds(N_TOK + tb, TPT)], w1_v)
    for c in range(TPT // 8):
        pltpu.async_copy(ys_hbm.at[p0_v.at[pl.ds(c * 8, 8)]], rowsA, sem).wait()
        pltpu.async_copy(ys_hbm.at[p1_v.at[pl.ds(c * 8, 8)]], rowsB, sem).wait()
        w0c = w0_v[pl.ds((c // 2) * 16, 16)]
        w1c = w1_v[pl.ds((c // 2) * 16, 16)]
        wa_s = [w0c[(c % 2) * 8 + j] for j in range(8)]
        wb_s = [w1c[(c % 2) * 8 + j] for j in range(8)]

        def addloop(r, _):
            for j in range(8):
                acc_v[j, pl.ds(r * 16, 16)] = (
                    rowsA[j, pl.ds(r * 16, 16)] * wa_s[j]
                    + rowsB[j, pl.ds(r * 16, 16)] * wb_s[j])
            return 0
        lax.fori_loop(0, FFN // 16, addloop, 0, unroll=2)
        pltpu.sync_copy(acc_v, out_hbm.at[pl.ds(tb + c * 8, 8)])


def _combine(pos, wgt, ys):
    return pl.kernel(
        _combine_body,
        out_type=jax.ShapeDtypeStruct((N_TOK, FFN), jnp.float32),
        mesh=plsc.VectorSubcoreMesh(core_axis_name="c", subcore_axis_name="s"),
        scratch_types=[
            pltpu.VMEM((TPT,), jnp.int32),
            pltpu.VMEM((TPT,), jnp.int32),
            pltpu.VMEM((TPT,), jnp.float32),
            pltpu.VMEM((TPT,), jnp.float32),
            pltpu.VMEM((8, FFN), jnp.float32),
            pltpu.VMEM((8, FFN), jnp.float32),
            pltpu.VMEM((8, FFN), jnp.float32),
            pltpu.SemaphoreType.DMA,
        ],
    )(pos, wgt, ys)


def kernel(hidden_states, Wg, W1, W2, W3):
    b, ch, h, w = hidden_states.shape
    x = jnp.transpose(hidden_states, (0, 2, 3, 1)).reshape(-1, ch)
    ei, ew = _router(x, Wg)
    eid = ei.reshape(NA)
    wgt = ew.reshape(NA)
    stok, pos, sexp, nbv = _sort(eid)
    xs = _gather(stok, x)
    ys = _mlp(sexp, nbv, xs, W1, W2, W3)
    out_flat = _combine(pos, wgt, ys)
    out = out_flat.reshape(b, h, w, FFN)
    return jnp.transpose(out, (0, 3, 1, 2))


# speedup vs baseline: 1.1454x; 1.1454x over previous
import functools

import jax
import jax.numpy as jnp
from jax import lax
from jax.experimental import pallas as pl
from jax.experimental.pallas import tpu as pltpu
from jax.experimental.pallas import tpu_sc as plsc

HIDDEN = 768
FFN = 3072
E = 8
N_TOK = 1024
NA = 2 * N_TOK
TBR = 128
NS = 24
NW = 32
GPT = (NS * TBR) // NW
TPT = N_TOK // NW


def _router_body(x_ref, wg_ref, ei_ref, ew_ref):
    x = x_ref[...]
    logits = lax.dot_general(x, wg_ref[...], (((1,), (1,)), ((), ())),
                             preferred_element_type=jnp.float32)
    m = jax.nn.softmax(logits, axis=-1)
    i1 = jnp.argmax(m, axis=-1).astype(jnp.int32)
    w1 = jnp.max(m, axis=-1)
    col = lax.broadcasted_iota(jnp.int32, m.shape, 1)
    m2 = jnp.where(col == i1[:, None], -jnp.inf, m)
    i2 = jnp.argmax(m2, axis=-1).astype(jnp.int32)
    w2 = jnp.max(m2, axis=-1)
    d = w1 + w2
    ei_ref[0, :] = i1
    ei_ref[1, :] = i2
    ew_ref[0, :] = w1 / d
    ew_ref[1, :] = w2 / d


def _router(x, Wg):
    return pl.pallas_call(
        _router_body,
        out_shape=(
            jax.ShapeDtypeStruct((2, N_TOK), jnp.int32),
            jax.ShapeDtypeStruct((2, N_TOK), jnp.float32),
        ),
    )(x, Wg)


def _sort_body(eid_hbm, stok_hbm, pos_hbm, sexp_hbm, nb_hbm,
               eid_v, stok_v, sa_v, pos_v, sexp_v, nb_v):
    cid = lax.axis_index("c")
    sid = lax.axis_index("s")

    @pl.when(jnp.logical_and(cid == 0, sid == 0))
    def _():
        pltpu.sync_copy(eid_hbm, eid_v)
        ioto = lax.iota(jnp.int32, 16)
        zeros = jnp.zeros((16,), jnp.int32)

        def initloop(i, _):
            stok_v[pl.ds(i * 16, 16)] = zeros
            sa_v[pl.ds(i * 16, 16)] = zeros + NA
            return 0
        lax.fori_loop(0, (NS * TBR + 16) // 16, initloop, 0)

        def histloop(i, cnts):
            v = eid_v[pl.ds(i * 16, 16)]
            return tuple(
                cnts[e] + plsc.all_reduce_population_count(v == e)
                for e in range(E))
        cnts = lax.fori_loop(
            0, NA // 16, histloop,
            tuple(jnp.zeros((16,), jnp.int32) for _ in range(E)))
        counts = [cnts[e][0] for e in range(E)]
        nbs = [(counts[e] + (TBR - 1)) >> 7 for e in range(E)]
        starts = []
        acc = jnp.int32(0)
        for e in range(E):
            starts.append(acc)
            acc = acc + nbs[e]
        num_blocks = acc
        ends = [starts[e] + nbs[e] for e in range(E)]

        for r in range(2):
            sl = ioto + r * 16
            ecnt = zeros
            for e in range(E):
                ecnt = ecnt + jnp.where(ends[e] <= sl, 1, 0)
            sexp_v[pl.ds(r * 16, 16)] = jnp.minimum(ecnt, E - 1)
        nb_v[...] = zeros + num_blocks

        for e in range(E):
            def p2loop(i, c, e=e):
                v = eid_v[pl.ds(i * 16, 16)]
                a_vec = ioto + i * 16
                m = v == e
                plsc.store_compressed(stok_v.at[pl.ds(c, 16)],
                                      a_vec & (N_TOK - 1), mask=m)
                plsc.store_compressed(sa_v.at[pl.ds(c, 16)], a_vec, mask=m)
                return c + plsc.all_reduce_population_count(m)[0]
            lax.fori_loop(0, NA // 16, p2loop, starts[e] * TBR)

        def invloop(j, _):
            av = sa_v[pl.ds(j * 16, 16)]
            plsc.store_scatter(pos_v, [av], ioto + j * 16)
            return 0
        lax.fori_loop(0, (NS * TBR) // 16, invloop, 0)

        pltpu.sync_copy(stok_v.at[pl.ds(0, NS * TBR)], stok_hbm)
        pltpu.sync_copy(pos_v.at[pl.ds(0, NA)], pos_hbm)
        pltpu.sync_copy(sexp_v, sexp_hbm)
        pltpu.sync_copy(nb_v, nb_hbm)


def _sort(eid):
    return pl.kernel(
        _sort_body,
        out_type=(
            jax.ShapeDtypeStruct((NS * TBR,), jnp.int32),
            jax.ShapeDtypeStruct((NA,), jnp.int32),
            jax.ShapeDtypeStruct((32,), jnp.int32),
            jax.ShapeDtypeStruct((16,), jnp.int32),
        ),
        mesh=plsc.VectorSubcoreMesh(core_axis_name="c", subcore_axis_name="s"),
        compiler_params=pltpu.CompilerParams(needs_layout_passes=False),
        scratch_types=[
            pltpu.VMEM((NA,), jnp.int32),
            pltpu.VMEM((NS * TBR + 16,), jnp.int32),
            pltpu.VMEM((NS * TBR + 16,), jnp.int32),
            pltpu.VMEM((NA + 16,), jnp.int32),
            pltpu.VMEM((32,), jnp.int32),
            pltpu.VMEM((16,), jnp.int32),
        ],
    )(eid)


def _gather_body(stok_hbm, nb_hbm, x_hbm, xs_hbm, idx_v, rows_v, nb_v, sem):
    wid = lax.axis_index("s") * 2 + lax.axis_index("c")
    base = wid * GPT
    pltpu.sync_copy(nb_hbm, nb_v)
    nrows = nb_v[pl.ds(0, 16)][0] * TBR

    @pl.when(base < nrows)
    def _():
        pltpu.sync_copy(stok_hbm.at[pl.ds(base, GPT)], idx_v)
        copies = [
            pltpu.async_copy(x_hbm.at[idx_v.at[pl.ds(k * 8, 8)]],
                             rows_v.at[pl.ds(k * 8, 8)], sem)
            for k in range(GPT // 8)
        ]
        for cp in copies:
            cp.wait()
        pltpu.sync_copy(rows_v, xs_hbm.at[pl.ds(base, GPT)])


def _gather(stok, nbv, x):
    return pl.kernel(
        _gather_body,
        out_type=jax.ShapeDtypeStruct((NS * TBR, HIDDEN), jnp.float32),
        mesh=plsc.VectorSubcoreMesh(core_axis_name="c", subcore_axis_name="s"),
        scratch_types=[
            pltpu.VMEM((GPT,), jnp.int32),
            pltpu.VMEM((GPT, HIDDEN), jnp.float32),
            pltpu.VMEM((16,), jnp.int32),
            pltpu.SemaphoreType.DMA,
        ],
    )(stok, nbv, x)


def _mlp_body(sexp_ref, nb_ref, xs_ref, w1_ref, w2_ref, w3_ref, ys_ref):
    s = pl.program_id(0)

    @pl.when(s < nb_ref[0])
    def _():
        x = xs_ref[...]
        h1 = jnp.maximum(
            lax.dot_general(x, w1_ref[0], (((1,), (1,)), ((), ())),
                            preferred_element_type=jnp.float32), 0.0)
        h2 = jnp.maximum(
            lax.dot_general(h1, w2_ref[0], (((1,), (1,)), ((), ())),
                            preferred_element_type=jnp.float32), 0.0)
        ys_ref[...] = lax.dot_general(h2, w3_ref[0], (((1,), (1,)), ((), ())),
                                      preferred_element_type=jnp.float32)


def _mlp(sexp, nbv, xs, W1, W2, W3):
    grid_spec = pltpu.PrefetchScalarGridSpec(
        num_scalar_prefetch=2,
        grid=(NS,),
        in_specs=[
            pl.BlockSpec((TBR, HIDDEN), lambda s, se, nb: (s, 0)),
            pl.BlockSpec((1, HIDDEN, HIDDEN), lambda s, se, nb: (se[s], 0, 0)),
            pl.BlockSpec((1, HIDDEN, HIDDEN), lambda s, se, nb: (se[s], 0, 0)),
            pl.BlockSpec((1, FFN, HIDDEN), lambda s, se, nb: (se[s], 0, 0)),
        ],
        out_specs=pl.BlockSpec((TBR, FFN), lambda s, se, nb: (s, 0)),
    )
    return pl.pallas_call(
        _mlp_body,
        grid_spec=grid_spec,
        out_shape=jax.ShapeDtypeStruct((NS * TBR, FFN), jnp.float32),
        compiler_params=pltpu.CompilerParams(
            dimension_semantics=("arbitrary",),
        ),
    )(sexp, nbv, xs, W1, W2, W3)


_CCH = 8
_NCH = TPT // _CCH


def _combine_body(pos_hbm, wgt_hbm, ys_hbm, out_hbm,
                  p0_v, p1_v, w0_v, w1_v,
                  rA0, rB0, rA1, rB1, acc0, sem, osem):
    wid = lax.axis_index("s") * 2 + lax.axis_index("c")
    tb = wid * TPT
    pltpu.sync_copy(pos_hbm.at[pl.ds(tb, TPT)], p0_v)
    pltpu.sync_copy(pos_hbm.at[pl.ds(N_TOK + tb, TPT)], p1_v)
    pltpu.sync_copy(wgt_hbm.at[pl.ds(tb, TPT)], w0_v)
    pltpu.sync_copy(wgt_hbm.at[pl.ds(N_TOK + tb, TPT)], w1_v)
    w0a = w0_v[pl.ds(0, 16)]
    w0b = w0_v[pl.ds(16, 16)]
    w1a = w1_v[pl.ds(0, 16)]
    w1b = w1_v[pl.ds(16, 16)]
    rows = [(rA0, rB0), (rA1, rB1)]

    def fire(c, buf):
        A, B = rows[buf]
        ca = pltpu.async_copy(ys_hbm.at[p0_v.at[pl.ds(c * _CCH, _CCH)]], A, sem)
        cb = pltpu.async_copy(ys_hbm.at[p1_v.at[pl.ds(c * _CCH, _CCH)]], B, sem)
        return ca, cb

    pend = fire(0, 0)
    ocopy = None
    for c in range(_NCH):
        nxt = fire(c + 1, (c + 1) % 2) if c + 1 < _NCH else None
        pend[0].wait()
        pend[1].wait()
        A, B = rows[c % 2]
        acc = acc0
        if ocopy is not None:
            ocopy.wait()
        wa = [(w0a if c * _CCH + j < 16 else w0b)[(c * _CCH + j) % 16]
              for j in range(_CCH)]
        wb = [(w1a if c * _CCH + j < 16 else w1b)[(c * _CCH + j) % 16]
              for j in range(_CCH)]

        def addloop(r, _, A=A, B=B, acc=acc, wa=wa, wb=wb):
            for j in range(_CCH):
                acc[j, pl.ds(r * 16, 16)] = (
                    A[j, pl.ds(r * 16, 16)] * wa[j]
                    + B[j, pl.ds(r * 16, 16)] * wb[j])
            return 0
        lax.fori_loop(0, FFN // 16, addloop, 0, unroll=4)
        ocopy = pltpu.async_copy(
            acc, out_hbm.at[pl.ds(tb + c * _CCH, _CCH)], osem)
        pend = nxt
    ocopy.wait()


def _combine(pos, wgt, ys):
    return pl.kernel(
        _combine_body,
        out_type=jax.ShapeDtypeStruct((N_TOK, FFN), jnp.float32),
        mesh=plsc.VectorSubcoreMesh(core_axis_name="c", subcore_axis_name="s"),
        scratch_types=[
            pltpu.VMEM((TPT,), jnp.int32),
            pltpu.VMEM((TPT,), jnp.int32),
            pltpu.VMEM((TPT,), jnp.float32),
            pltpu.VMEM((TPT,), jnp.float32),
            pltpu.VMEM((_CCH, FFN), jnp.float32),
            pltpu.VMEM((_CCH, FFN), jnp.float32),
            pltpu.VMEM((_CCH, FFN), jnp.float32),
            pltpu.VMEM((_CCH, FFN), jnp.float32),
            pltpu.VMEM((_CCH, FFN), jnp.float32),
            pltpu.SemaphoreType.DMA,
            pltpu.SemaphoreType.DMA,
        ],
    )(pos, wgt, ys)


def kernel(hidden_states, Wg, W1, W2, W3):
    b, ch, h, w = hidden_states.shape
    x = jnp.transpose(hidden_states, (0, 2, 3, 1)).reshape(-1, ch)
    ei, ew = _router(x, Wg)
    eid = ei.reshape(NA)
    wgt = ew.reshape(NA)
    stok, pos, sexp, nbv = _sort(eid)
    xs = _gather(stok, nbv, x)
    ys = _mlp(sexp, nbv, xs, W1, W2, W3)
    out_flat = _combine(pos, wgt, ys)
    out = out_flat.reshape(b, h, w, FFN)
    return jnp.transpose(out, (0, 3, 1, 2))
